# baseline (speedup 1.0000x reference)
import jax
import jax.numpy as jnp
from jax.experimental import pallas as pl


def _mlp_block(src_ref, dst_ref, w1a_ref, w1b_ref, b1_ref, w2t_ref, b2_ref,
               out_ref):
    h = jnp.dot(src_ref[...], w1a_ref[...], preferred_element_type=jnp.float32)
    h = h + jnp.dot(dst_ref[...], w1b_ref[...],
                    preferred_element_type=jnp.float32)
    h = jnp.maximum(h + b1_ref[...], 0.0)
    out_ref[...] = (jnp.sum(h * w2t_ref[...], axis=1, keepdims=True)
                    + b2_ref[...])


def kernel(src, dst, W1, b1, W2, b2):
    E, F = src.shape
    H = W1.shape[1]
    BLOCK = 2560
    grid = (E // BLOCK,)
    return pl.pallas_call(
        _mlp_block,
        grid=grid,
        in_specs=[
            pl.BlockSpec((BLOCK, F), lambda i: (i, 0)),
            pl.BlockSpec((BLOCK, F), lambda i: (i, 0)),
            pl.BlockSpec((F, H), lambda i: (0, 0)),
            pl.BlockSpec((F, H), lambda i: (0, 0)),
            pl.BlockSpec((1, H), lambda i: (0, 0)),
            pl.BlockSpec((1, H), lambda i: (0, 0)),
            pl.BlockSpec((1, 1), lambda i: (0, 0)),
        ],
        out_specs=pl.BlockSpec((BLOCK, 1), lambda i: (i, 0)),
        out_shape=jax.ShapeDtypeStruct((E, 1), jnp.float32),
    )(src, dst, W1[:F], W1[F:], b1.reshape(1, H), W2.reshape(1, H),
      b2.reshape(1, 1))
